# baseline (speedup 1.0000x reference)
import functools

import jax
import jax.numpy as jnp
import numpy as np
from jax import lax
from jax.experimental import pallas as pl
from jax.experimental.pallas import tpu as pltpu
from jax.experimental.pallas import tpu_sc as plsc

G = 1000
SIZE = 0.001
DELTAZ = 0.001
COEF_A = 0.0136
COEF_B = 0.038
SQRT12 = np.sqrt(12.0)

LANE = 128
R = 8192
NPAD = R * LANE

NC, NS = 2, 16
NW = NC * NS
CHUNK = NPAD // NW
SUB = 4096
NSUB = CHUNK // SUB
GSZ = 128
VL = 16


def _sc_gather_body(x_hbm, y_hbm, tab_hbm, out_hbm, xbuf, ybuf, idxbuf, gbuf, sem):
    wid = lax.axis_index("s") * NC + lax.axis_index("c")

    def sub_body(s, carry):
        base = wid * CHUNK + s * SUB
        pltpu.sync_copy(x_hbm.at[pl.ds(base, SUB)], xbuf)
        pltpu.sync_copy(y_hbm.at[pl.ds(base, SUB)], ybuf)

        def idx_body(i, c):
            xv = xbuf[pl.ds(i * VL, VL)]
            yv = ybuf[pl.ds(i * VL, VL)]
            ix = jnp.clip((xv / SIZE).astype(jnp.int32), 0, G - 1)
            iy = jnp.clip((yv / SIZE).astype(jnp.int32), 0, G - 1)
            idxbuf[pl.ds(i * VL, VL)] = ix * G + iy
            return c

        lax.fori_loop(0, SUB // VL, idx_body, 0, unroll=4)

        def fire(k, c):
            pltpu.async_copy(
                tab_hbm.at[idxbuf.at[pl.ds(k * GSZ, GSZ)]],
                gbuf.at[pl.ds(k * GSZ, GSZ)],
                sem,
            )
            return c

        lax.fori_loop(0, SUB // GSZ, fire, 0)
        pltpu.make_async_copy(tab_hbm.at[pl.ds(0, SUB)], gbuf, sem).wait()
        pltpu.sync_copy(gbuf, out_hbm.at[pl.ds(base, SUB)])
        return carry

    lax.fori_loop(0, NSUB, sub_body, 0)


def _sc_gather(x, y, table_flat):
    mesh = plsc.VectorSubcoreMesh(core_axis_name="c", subcore_axis_name="s")
    k = pl.kernel(
        _sc_gather_body,
        out_type=jax.ShapeDtypeStruct((NPAD,), jnp.float32),
        mesh=mesh,
        scratch_types=[
            pltpu.VMEM((SUB,), jnp.float32),
            pltpu.VMEM((SUB,), jnp.float32),
            pltpu.VMEM((SUB,), jnp.int32),
            pltpu.VMEM((SUB,), jnp.float32),
            pltpu.SemaphoreType.DMA,
        ],
    )
    return k(x, y, table_flat)


def _tc_physics_body(xy_ref, tt_ref, z1_ref, z2_ref, mom_ref, x0_ref, out_ref):
    x = xy_ref[0]
    y = xy_ref[1]
    mask = (x >= 0.0) & (x < 1.0) & (y >= 0.0) & (y < 1.0)
    mom = mom_ref[...]
    x0 = x0_ref[...]
    a_over_p = COEF_A / mom
    for c in (0, 1):
        t = tt_ref[c]
        z1 = z1_ref[c]
        z2 = z2_ref[c]
        cos_t = jnp.cos(t)
        flight = DELTAZ / cos_t
        n_x0 = flight / x0
        theta0 = a_over_p * jnp.sqrt(n_x0)
        theta0 = theta0 * (1.0 + COEF_B * jnp.log(n_x0))
        dtheta = z1 * theta0
        dxy = flight * jnp.sin(theta0) * (z1 / SQRT12 + z2 / 2.0)
        dxy_vol = dxy * cos_t
        out_ref[c] = jnp.where(mask, dtheta, 0.0)
        out_ref[c + 2] = jnp.where(mask, dxy_vol, 0.0)


def _tc_physics(xyT, ttT, z1T, z2T, momp, x0r):
    BR = 512
    grid = (R // BR,)
    pair = pl.BlockSpec((2, BR, LANE), lambda i: (0, i, 0))
    single = pl.BlockSpec((BR, LANE), lambda i: (i, 0))
    return pl.pallas_call(
        _tc_physics_body,
        grid=grid,
        in_specs=[pair, pair, pair, pair, single, single],
        out_specs=pl.BlockSpec((4, BR, LANE), lambda i: (0, i, 0)),
        out_shape=jax.ShapeDtypeStruct((4, R, LANE), jnp.float32),
        compiler_params=pltpu.CompilerParams(
            dimension_semantics=("arbitrary",),
        ),
    )(xyT, ttT, z1T, z2T, momp, x0r)


def kernel(xy, theta_xy, mom, z1, z2, rad_length):
    n = xy.shape[0]
    pad = NPAD - n

    def prep_pair(a):
        return jnp.pad(a, ((0, pad), (0, 0))).T.reshape(2, R, LANE)

    xyT = prep_pair(xy)
    ttT = prep_pair(theta_xy)
    z1T = prep_pair(z1)
    z2T = prep_pair(z2)
    momp = jnp.pad(mom, (0, pad)).reshape(R, LANE)

    x0_flat = _sc_gather(
        xyT[0].reshape(NPAD), xyT[1].reshape(NPAD), rad_length.reshape(G * G)
    )
    x0r = x0_flat.reshape(R, LANE)

    out4 = _tc_physics(xyT, ttT, z1T, z2T, momp, x0r)
    return out4.reshape(4, NPAD)[:, :n].T


# speedup vs baseline: 1.0606x; 1.0606x over previous
import functools

import jax
import jax.numpy as jnp
import numpy as np
from jax import lax
from jax.experimental import pallas as pl
from jax.experimental.pallas import tpu as pltpu
from jax.experimental.pallas import tpu_sc as plsc

G = 1000
SIZE = 0.001
DELTAZ = 0.001
COEF_A = 0.0136
COEF_B = 0.038
SQRT12 = np.sqrt(12.0)

LANE = 128
R = 8192
NPAD = R * LANE

NC, NS = 2, 16
NW = NC * NS
CHUNK = NPAD // NW
SUB = 4096
NSUB = CHUNK // SUB
GSZ = 128
VL = 16


def _sc_gather_body(
    x_hbm, y_hbm, tab_hbm, out_hbm, xa, ya, xb, yb, idxbuf, gbuf, sem_s, sem_g
):
    wid = lax.axis_index("s") * NC + lax.axis_index("c")
    cbase = wid * CHUNK
    xs, ys = (xa, xb), (ya, yb)
    pltpu.async_copy(x_hbm.at[pl.ds(cbase, SUB)], xa, sem_s)
    pltpu.async_copy(y_hbm.at[pl.ds(cbase, SUB)], ya, sem_s)
    for s in range(NSUB):
        xc, yc = xs[s % 2], ys[s % 2]
        pltpu.make_async_copy(x_hbm.at[pl.ds(0, SUB)], xc, sem_s).wait()
        pltpu.make_async_copy(x_hbm.at[pl.ds(0, SUB)], yc, sem_s).wait()
        if s + 1 < NSUB:
            nb = cbase + (s + 1) * SUB
            pltpu.async_copy(x_hbm.at[pl.ds(nb, SUB)], xs[(s + 1) % 2], sem_s)
            pltpu.async_copy(y_hbm.at[pl.ds(nb, SUB)], ys[(s + 1) % 2], sem_s)

        def jbody(j, c, s=s, xc=xc, yc=yc):
            off = s * SUB + j * GSZ
            for i in range(GSZ // VL):
                xv = xc[pl.ds(j * GSZ + i * VL, VL)]
                yv = yc[pl.ds(j * GSZ + i * VL, VL)]
                ix = jnp.clip((xv / SIZE).astype(jnp.int32), 0, G - 1)
                iy = jnp.clip((yv / SIZE).astype(jnp.int32), 0, G - 1)
                idxbuf[pl.ds(off + i * VL, VL)] = ix * G + iy
            pltpu.async_copy(
                tab_hbm.at[idxbuf.at[pl.ds(off, GSZ)]],
                gbuf.at[pl.ds(off, GSZ)],
                sem_g,
            )
            return c

        lax.fori_loop(0, SUB // GSZ, jbody, 0)
    pltpu.make_async_copy(tab_hbm.at[pl.ds(0, CHUNK)], gbuf, sem_g).wait()
    pltpu.sync_copy(gbuf, out_hbm.at[pl.ds(cbase, CHUNK)])


def _sc_gather(x, y, table_flat):
    mesh = plsc.VectorSubcoreMesh(core_axis_name="c", subcore_axis_name="s")
    k = pl.kernel(
        _sc_gather_body,
        out_type=jax.ShapeDtypeStruct((NPAD,), jnp.float32),
        mesh=mesh,
        scratch_types=[
            pltpu.VMEM((SUB,), jnp.float32),
            pltpu.VMEM((SUB,), jnp.float32),
            pltpu.VMEM((SUB,), jnp.float32),
            pltpu.VMEM((SUB,), jnp.float32),
            pltpu.VMEM((CHUNK,), jnp.int32),
            pltpu.VMEM((CHUNK,), jnp.float32),
            pltpu.SemaphoreType.DMA,
            pltpu.SemaphoreType.DMA,
        ],
    )
    return k(x, y, table_flat)


def _tc_physics_body(xy_ref, tt_ref, z1_ref, z2_ref, mom_ref, x0_ref, out_ref):
    x = xy_ref[0]
    y = xy_ref[1]
    mask = (x >= 0.0) & (x < 1.0) & (y >= 0.0) & (y < 1.0)
    mom = mom_ref[...]
    x0 = x0_ref[...]
    a_over_p = COEF_A / mom
    for c in (0, 1):
        t = tt_ref[c]
        z1 = z1_ref[c]
        z2 = z2_ref[c]
        cos_t = jnp.cos(t)
        flight = DELTAZ / cos_t
        n_x0 = flight / x0
        theta0 = a_over_p * jnp.sqrt(n_x0)
        theta0 = theta0 * (1.0 + COEF_B * jnp.log(n_x0))
        dtheta = z1 * theta0
        dxy = flight * jnp.sin(theta0) * (z1 / SQRT12 + z2 / 2.0)
        dxy_vol = dxy * cos_t
        out_ref[c] = jnp.where(mask, dtheta, 0.0)
        out_ref[c + 2] = jnp.where(mask, dxy_vol, 0.0)


def _tc_physics(xyT, ttT, z1T, z2T, momp, x0r):
    BR = 512
    grid = (R // BR,)
    pair = pl.BlockSpec((2, BR, LANE), lambda i: (0, i, 0))
    single = pl.BlockSpec((BR, LANE), lambda i: (i, 0))
    return pl.pallas_call(
        _tc_physics_body,
        grid=grid,
        in_specs=[pair, pair, pair, pair, single, single],
        out_specs=pl.BlockSpec((4, BR, LANE), lambda i: (0, i, 0)),
        out_shape=jax.ShapeDtypeStruct((4, R, LANE), jnp.float32),
        compiler_params=pltpu.CompilerParams(
            dimension_semantics=("arbitrary",),
        ),
    )(xyT, ttT, z1T, z2T, momp, x0r)


def kernel(xy, theta_xy, mom, z1, z2, rad_length):
    n = xy.shape[0]
    pad = NPAD - n

    def prep_pair(a):
        return jnp.pad(a, ((0, pad), (0, 0))).T.reshape(2, R, LANE)

    xyT = prep_pair(xy)
    ttT = prep_pair(theta_xy)
    z1T = prep_pair(z1)
    z2T = prep_pair(z2)
    momp = jnp.pad(mom, (0, pad)).reshape(R, LANE)

    x0_flat = _sc_gather(
        xyT[0].reshape(NPAD), xyT[1].reshape(NPAD), rad_length.reshape(G * G)
    )
    x0r = x0_flat.reshape(R, LANE)

    out4 = _tc_physics(xyT, ttT, z1T, z2T, momp, x0r)
    return out4.reshape(4, NPAD)[:, :n].T


# speedup vs baseline: 1.0868x; 1.0246x over previous
import functools

import jax
import jax.numpy as jnp
import numpy as np
from jax import lax
from jax.experimental import pallas as pl
from jax.experimental.pallas import tpu as pltpu
from jax.experimental.pallas import tpu_sc as plsc

G = 1000
SIZE = 0.001
DELTAZ = 0.001
COEF_A = 0.0136
COEF_B = 0.038
SQRT12 = np.sqrt(12.0)

LANE = 128
R = 8192
NPAD = R * LANE

NC, NS = 2, 16
NW = NC * NS
CHUNK = NPAD // NW
SUB = 4096
NSUB = CHUNK // SUB
GSZ = 128
VL = 16


def _sc_gather_body(idx_hbm, tab_hbm, out_hbm, idxbuf, gbuf, sem_g):
    wid = lax.axis_index("s") * NC + lax.axis_index("c")
    cbase = wid * CHUNK
    pltpu.sync_copy(idx_hbm.at[pl.ds(cbase, CHUNK)], idxbuf)

    def fire(k, c):
        pltpu.async_copy(
            tab_hbm.at[idxbuf.at[pl.ds(k * GSZ, GSZ)]],
            gbuf.at[pl.ds(k * GSZ, GSZ)],
            sem_g,
        )
        return c

    lax.fori_loop(0, CHUNK // GSZ, fire, 0)
    pltpu.make_async_copy(tab_hbm.at[pl.ds(0, CHUNK)], gbuf, sem_g).wait()
    pltpu.sync_copy(gbuf, out_hbm.at[pl.ds(cbase, CHUNK)])


def _sc_gather(idx_flat, table_flat):
    mesh = plsc.VectorSubcoreMesh(core_axis_name="c", subcore_axis_name="s")
    k = pl.kernel(
        _sc_gather_body,
        out_type=jax.ShapeDtypeStruct((NPAD,), jnp.float32),
        mesh=mesh,
        scratch_types=[
            pltpu.VMEM((CHUNK,), jnp.int32),
            pltpu.VMEM((CHUNK,), jnp.float32),
            pltpu.SemaphoreType.DMA,
        ],
    )
    return k(idx_flat, table_flat)


def _tc_idx_body(xy_ref, idx_ref):
    x = xy_ref[0]
    y = xy_ref[1]
    ix = jnp.clip((x / SIZE).astype(jnp.int32), 0, G - 1)
    iy = jnp.clip((y / SIZE).astype(jnp.int32), 0, G - 1)
    idx_ref[...] = ix * G + iy


def _tc_idx(xyT):
    BR = 1024
    return pl.pallas_call(
        _tc_idx_body,
        grid=(R // BR,),
        in_specs=[pl.BlockSpec((2, BR, LANE), lambda i: (0, i, 0))],
        out_specs=pl.BlockSpec((BR, LANE), lambda i: (i, 0)),
        out_shape=jax.ShapeDtypeStruct((R, LANE), jnp.int32),
        compiler_params=pltpu.CompilerParams(
            dimension_semantics=("arbitrary",),
        ),
    )(xyT)


def _tc_physics_body(xy_ref, tt_ref, z1_ref, z2_ref, mom_ref, x0_ref, out_ref):
    x = xy_ref[0]
    y = xy_ref[1]
    mask = (x >= 0.0) & (x < 1.0) & (y >= 0.0) & (y < 1.0)
    mom = mom_ref[...]
    x0 = x0_ref[...]
    a_over_p = COEF_A / mom
    for c in (0, 1):
        t = tt_ref[c]
        z1 = z1_ref[c]
        z2 = z2_ref[c]
        cos_t = jnp.cos(t)
        flight = DELTAZ / cos_t
        n_x0 = flight / x0
        theta0 = a_over_p * jnp.sqrt(n_x0)
        theta0 = theta0 * (1.0 + COEF_B * jnp.log(n_x0))
        dtheta = z1 * theta0
        dxy = flight * jnp.sin(theta0) * (z1 / SQRT12 + z2 / 2.0)
        dxy_vol = dxy * cos_t
        out_ref[c] = jnp.where(mask, dtheta, 0.0)
        out_ref[c + 2] = jnp.where(mask, dxy_vol, 0.0)


def _tc_physics(xyT, ttT, z1T, z2T, momp, x0r):
    BR = 512
    grid = (R // BR,)
    pair = pl.BlockSpec((2, BR, LANE), lambda i: (0, i, 0))
    single = pl.BlockSpec((BR, LANE), lambda i: (i, 0))
    return pl.pallas_call(
        _tc_physics_body,
        grid=grid,
        in_specs=[pair, pair, pair, pair, single, single],
        out_specs=pl.BlockSpec((4, BR, LANE), lambda i: (0, i, 0)),
        out_shape=jax.ShapeDtypeStruct((4, R, LANE), jnp.float32),
        compiler_params=pltpu.CompilerParams(
            dimension_semantics=("arbitrary",),
        ),
    )(xyT, ttT, z1T, z2T, momp, x0r)


def kernel(xy, theta_xy, mom, z1, z2, rad_length):
    n = xy.shape[0]
    pad = NPAD - n

    def prep_pair(a):
        return jnp.pad(a, ((0, pad), (0, 0))).T.reshape(2, R, LANE)

    xyT = prep_pair(xy)
    ttT = prep_pair(theta_xy)
    z1T = prep_pair(z1)
    z2T = prep_pair(z2)
    momp = jnp.pad(mom, (0, pad)).reshape(R, LANE)

    idx = _tc_idx(xyT)
    x0_flat = _sc_gather(idx.reshape(NPAD), rad_length.reshape(G * G))
    x0r = x0_flat.reshape(R, LANE)

    out4 = _tc_physics(xyT, ttT, z1T, z2T, momp, x0r)
    return out4.reshape(4, NPAD)[:, :n].T
